# baseline (speedup 1.0000x reference)
import functools

import jax
import jax.numpy as jnp
from jax import lax
from jax.experimental import pallas as pl
from jax.experimental.pallas import tpu as pltpu
from jax.experimental.pallas import tpu_sc as plsc

C = 4096
NC, NS, L = 2, 16, 16
SB = 32
W = 128
NSCB = 8
NG = W // L
R = 128
RSPLIT = 4
SPAN = C // RSPLIT
HALF = NSCB * W


def _sc_body(act_hbm, hold_hbm, dom_hbm, outhold_hbm,
             act_v, hold_v, nh_v, part_v, dbuf, shared,
             sem_in0, sem_in1):
    core = lax.axis_index("c")
    sub = lax.axis_index("s")
    sem_in = (sem_in0, sem_in1)
    band = core * (NSCB // NC) + sub // RSPLIT
    rh = sub % RSPLIT
    n0 = band * W
    m_base = rh * SPAN

    pltpu.sync_copy(act_hbm.at[pl.ds(n0, W)], act_v)
    pltpu.sync_copy(hold_hbm.at[:], hold_v.at[pl.ds(0, C)])

    a_g = [act_v[pl.ds(g * L, L)] for g in range(NG)]

    zero = jnp.zeros((L,), jnp.float32)
    carry = tuple(zero for _ in range(3 * NG))

    def in_copy(j, p):
        return pltpu.async_copy(
            dom_hbm.at[pl.ds(m_base + j * R, R), band], dbuf.at[p], sem_in[p])

    in_copy(0, 0)
    in_copy(1, 1)

    RU = 8
    NCH = R // RU
    NBH = SPAN // R

    def pair_body(i, carry):
        for p in (0, 1):
            j = 2 * i + p
            db = dbuf.at[p]
            pltpu.make_async_copy(
                dom_hbm.at[pl.ds(0, R), band], db, sem_in[p]).wait()
            m0 = m_base + j * R

            def chunk_body(cc, t, db=db, m0=m0):
                mb = cc * RU
                hvec = hold_v[pl.ds(m0 + mb, L)]
                t = list(t)
                for k in range(RU):
                    hv = jnp.full((L,), hvec[k], jnp.float32)
                    for g in range(NG):
                        d = db[mb + k, pl.ds(g * L, L)]
                        pr = d * hv
                        t0, t1, t2 = t[3 * g], t[3 * g + 1], t[3 * g + 2]
                        n0v = jnp.maximum(t0, pr)
                        r1 = jnp.minimum(t0, pr)
                        n1v = jnp.maximum(t1, r1)
                        r2 = jnp.minimum(t1, r1)
                        n2v = jnp.maximum(t2, r2)
                        t[3 * g], t[3 * g + 1], t[3 * g + 2] = n0v, n1v, n2v
                return tuple(t)

            carry = lax.fori_loop(0, NCH, chunk_body, tuple(carry))

            @pl.when(j + 2 < NBH)
            def _():
                in_copy(j + 2, p)

        return carry

    carry = lax.fori_loop(0, NBH // 2, pair_body, carry)

    for r in range(3):
        for g in range(NG):
            part_v[r, pl.ds(g * L, L)] = carry[3 * g + r]
    pltpu.sync_copy(part_v, shared.at[sub])
    plsc.subcore_barrier()

    @pl.when(rh == 0)
    def _():
        t = list(carry)
        for q in range(1, RSPLIT):
            pltpu.sync_copy(shared.at[sub + q], part_v)
            for g in range(NG):
                t0, t1, t2 = t[3 * g], t[3 * g + 1], t[3 * g + 2]
                for r in range(3):
                    pr = part_v[r, pl.ds(g * L, L)]
                    t0n = jnp.maximum(t0, pr)
                    r1 = jnp.minimum(t0, pr)
                    t1n = jnp.maximum(t1, r1)
                    r2 = jnp.minimum(t1, r1)
                    t2n = jnp.maximum(t2, r2)
                    t0, t1, t2 = t0n, t1n, t2n
                t[3 * g], t[3 * g + 1], t[3 * g + 2] = t0, t1, t2
        for g in range(NG):
            v0 = t[3 * g] * a_g[g]
            v1 = t[3 * g + 1] * a_g[g]
            v2 = t[3 * g + 2] * a_g[g]
            nh_v[pl.ds(g * L, L)] = (
                1.0 - (1.0 - v0) * (1.0 - v1) * (1.0 - v2))
        pltpu.sync_copy(nh_v, outhold_hbm.at[pl.ds(n0, W)])


_sc_call = functools.partial(
    pl.kernel,
    out_type=jax.ShapeDtypeStruct((HALF,), jnp.float32),
    mesh=plsc.VectorSubcoreMesh(
        core_axis_name="c", subcore_axis_name="s", num_cores=NC,
        num_subcores=NS),
    scratch_types=[
        pltpu.VMEM((W,), jnp.float32),
        pltpu.VMEM((C + L,), jnp.float32),
        pltpu.VMEM((W,), jnp.float32),
        pltpu.VMEM((3, W), jnp.float32),
        pltpu.VMEM((2, R, W), jnp.float32),
        pltpu.VMEM_SHARED((NS, 3, W), jnp.float32),
        pltpu.SemaphoreType.DMA,
        pltpu.SemaphoreType.DMA,
    ],
)(_sc_body)


TBM = 128
HS = SB - NSCB


def _tc_body(hold_smem, act_ref, dom_ref, out_ref, nh_ref, t_ref):
    gi = pl.program_id(0)
    i0 = gi * TBM
    A = 1.0 - act_ref[...]

    @pl.when(gi == 0)
    def _():
        t_ref[...] = jnp.zeros((3, HS, 128), jnp.float32)

    def slab(i, t):
        t0, t1, t2 = t
        h = hold_smem[i0 + i]
        d = dom_ref[i]
        p1 = d * A
        p2 = d * (1.0 - h)
        out_ref[i] = p1 + p2 - p1 * p2
        pr = d[NSCB:, :] * h
        n0v = jnp.maximum(t0, pr)
        r1 = jnp.minimum(t0, pr)
        n1v = jnp.maximum(t1, r1)
        r2 = jnp.minimum(t1, r1)
        n2v = jnp.maximum(t2, r2)
        return (n0v, n1v, n2v)

    t = lax.fori_loop(0, TBM, slab, (t_ref[0], t_ref[1], t_ref[2]))
    t_ref[0], t_ref[1], t_ref[2] = t

    @pl.when(gi == C // TBM - 1)
    def _():
        a = act_ref[NSCB:, :]
        v0 = t_ref[0] * a
        v1 = t_ref[1] * a
        v2 = t_ref[2] * a
        nh_ref[...] = 1.0 - (1.0 - v0) * (1.0 - v1) * (1.0 - v2)


_tc_call = pl.pallas_call(
    _tc_body,
    grid=(C // TBM,),
    in_specs=[
        pl.BlockSpec(memory_space=pltpu.SMEM),
        pl.BlockSpec((SB, 128), lambda i: (0, 0)),
        pl.BlockSpec((TBM, SB, 128), lambda i: (i, 0, 0)),
    ],
    out_specs=[
        pl.BlockSpec((TBM, SB, 128), lambda i: (i, 0, 0)),
        pl.BlockSpec((HS, 128), lambda i: (0, 0)),
    ],
    out_shape=[
        jax.ShapeDtypeStruct((C, SB, 128), jnp.float32),
        jax.ShapeDtypeStruct((HS, 128), jnp.float32),
    ],
    scratch_shapes=[pltpu.VMEM((3, HS, 128), jnp.float32)],
)


def kernel(action, holding, dominos):
    dom = dominos.reshape(C, SB, 128)
    act2 = action.reshape(SB, 128)
    out_dom, nh_right = _tc_call(holding, act2, dom)
    nh_left = _sc_call(action, holding, dom)
    next_holding = jnp.concatenate([nh_left, nh_right.reshape(-1)])
    return next_holding, out_dom.reshape(-1)


# speedup vs baseline: 1.0200x; 1.0200x over previous
import functools

import jax
import jax.numpy as jnp
from jax import lax
from jax.experimental import pallas as pl
from jax.experimental.pallas import tpu as pltpu
from jax.experimental.pallas import tpu_sc as plsc

C = 4096
NC, NS, L = 2, 16, 16
SB = 32
W = 128
NSCB = 16
NG = W // L
R = 128
RSPLIT = 2
SPAN = C // RSPLIT
HALF = NSCB * W


def _sc_body(act_hbm, hold_hbm, dom_hbm, outhold_hbm,
             act_v, hold_v, nh_v, part_v, dbuf, shared,
             sem_in0, sem_in1):
    core = lax.axis_index("c")
    sub = lax.axis_index("s")
    sem_in = (sem_in0, sem_in1)
    band = core * (NSCB // NC) + sub // RSPLIT
    rh = sub % RSPLIT
    n0 = band * W
    m_base = rh * SPAN

    pltpu.sync_copy(act_hbm.at[pl.ds(n0, W)], act_v)
    pltpu.sync_copy(hold_hbm.at[:], hold_v.at[pl.ds(0, C)])

    a_g = [act_v[pl.ds(g * L, L)] for g in range(NG)]

    zero = jnp.zeros((L,), jnp.float32)
    carry = tuple(zero for _ in range(3 * NG))

    def in_copy(j, p):
        return pltpu.async_copy(
            dom_hbm.at[pl.ds(m_base + j * R, R), band], dbuf.at[p], sem_in[p])

    in_copy(0, 0)
    in_copy(1, 1)

    RU = 8
    NCH = R // RU
    NBH = SPAN // R

    def pair_body(i, carry):
        for p in (0, 1):
            j = 2 * i + p
            db = dbuf.at[p]
            pltpu.make_async_copy(
                dom_hbm.at[pl.ds(0, R), band], db, sem_in[p]).wait()
            m0 = m_base + j * R

            def chunk_body(cc, t, db=db, m0=m0):
                mb = cc * RU
                hvec = hold_v[pl.ds(m0 + mb, L)]
                t = list(t)
                for k in range(RU):
                    hv = jnp.full((L,), hvec[k], jnp.float32)
                    for g in range(NG):
                        d = db[mb + k, pl.ds(g * L, L)]
                        pr = d * hv
                        t0, t1, t2 = t[3 * g], t[3 * g + 1], t[3 * g + 2]
                        n0v = jnp.maximum(t0, pr)
                        r1 = jnp.minimum(t0, pr)
                        n1v = jnp.maximum(t1, r1)
                        r2 = jnp.minimum(t1, r1)
                        n2v = jnp.maximum(t2, r2)
                        t[3 * g], t[3 * g + 1], t[3 * g + 2] = n0v, n1v, n2v
                return tuple(t)

            carry = lax.fori_loop(0, NCH, chunk_body, tuple(carry))

            @pl.when(j + 2 < NBH)
            def _():
                in_copy(j + 2, p)

        return carry

    carry = lax.fori_loop(0, NBH // 2, pair_body, carry)

    for r in range(3):
        for g in range(NG):
            part_v[r, pl.ds(g * L, L)] = carry[3 * g + r]
    pltpu.sync_copy(part_v, shared.at[sub])
    plsc.subcore_barrier()

    @pl.when(rh == 0)
    def _():
        t = list(carry)
        for q in range(1, RSPLIT):
            pltpu.sync_copy(shared.at[sub + q], part_v)
            for g in range(NG):
                t0, t1, t2 = t[3 * g], t[3 * g + 1], t[3 * g + 2]
                for r in range(3):
                    pr = part_v[r, pl.ds(g * L, L)]
                    t0n = jnp.maximum(t0, pr)
                    r1 = jnp.minimum(t0, pr)
                    t1n = jnp.maximum(t1, r1)
                    r2 = jnp.minimum(t1, r1)
                    t2n = jnp.maximum(t2, r2)
                    t0, t1, t2 = t0n, t1n, t2n
                t[3 * g], t[3 * g + 1], t[3 * g + 2] = t0, t1, t2
        for g in range(NG):
            v0 = t[3 * g] * a_g[g]
            v1 = t[3 * g + 1] * a_g[g]
            v2 = t[3 * g + 2] * a_g[g]
            nh_v[pl.ds(g * L, L)] = (
                1.0 - (1.0 - v0) * (1.0 - v1) * (1.0 - v2))
        pltpu.sync_copy(nh_v, outhold_hbm.at[pl.ds(n0, W)])


_sc_call = functools.partial(
    pl.kernel,
    out_type=jax.ShapeDtypeStruct((HALF,), jnp.float32),
    mesh=plsc.VectorSubcoreMesh(
        core_axis_name="c", subcore_axis_name="s", num_cores=NC,
        num_subcores=NS),
    scratch_types=[
        pltpu.VMEM((W,), jnp.float32),
        pltpu.VMEM((C + L,), jnp.float32),
        pltpu.VMEM((W,), jnp.float32),
        pltpu.VMEM((3, W), jnp.float32),
        pltpu.VMEM((2, R, W), jnp.float32),
        pltpu.VMEM_SHARED((NS, 3, W), jnp.float32),
        pltpu.SemaphoreType.DMA,
        pltpu.SemaphoreType.DMA,
    ],
)(_sc_body)


TBM = 128
HS = SB - NSCB


def _tc_body(hold_smem, act_ref, dom_ref, out_ref, nh_ref, t_ref):
    gi = pl.program_id(0)
    i0 = gi * TBM
    A = 1.0 - act_ref[...]

    @pl.when(gi == 0)
    def _():
        t_ref[...] = jnp.zeros((3, HS, 128), jnp.float32)

    def slab(i, t):
        t0, t1, t2 = t
        h = hold_smem[i0 + i]
        d = dom_ref[i]
        p1 = d * A
        p2 = d * (1.0 - h)
        out_ref[i] = p1 + p2 - p1 * p2
        pr = d[NSCB:, :] * h
        n0v = jnp.maximum(t0, pr)
        r1 = jnp.minimum(t0, pr)
        n1v = jnp.maximum(t1, r1)
        r2 = jnp.minimum(t1, r1)
        n2v = jnp.maximum(t2, r2)
        return (n0v, n1v, n2v)

    t = lax.fori_loop(0, TBM, slab, (t_ref[0], t_ref[1], t_ref[2]))
    t_ref[0], t_ref[1], t_ref[2] = t

    @pl.when(gi == C // TBM - 1)
    def _():
        a = act_ref[NSCB:, :]
        v0 = t_ref[0] * a
        v1 = t_ref[1] * a
        v2 = t_ref[2] * a
        nh_ref[...] = 1.0 - (1.0 - v0) * (1.0 - v1) * (1.0 - v2)


_tc_call = pl.pallas_call(
    _tc_body,
    grid=(C // TBM,),
    in_specs=[
        pl.BlockSpec(memory_space=pltpu.SMEM),
        pl.BlockSpec((SB, 128), lambda i: (0, 0)),
        pl.BlockSpec((TBM, SB, 128), lambda i: (i, 0, 0)),
    ],
    out_specs=[
        pl.BlockSpec((TBM, SB, 128), lambda i: (i, 0, 0)),
        pl.BlockSpec((HS, 128), lambda i: (0, 0)),
    ],
    out_shape=[
        jax.ShapeDtypeStruct((C, SB, 128), jnp.float32),
        jax.ShapeDtypeStruct((HS, 128), jnp.float32),
    ],
    scratch_shapes=[pltpu.VMEM((3, HS, 128), jnp.float32)],
)


def kernel(action, holding, dominos):
    dom = dominos.reshape(C, SB, 128)
    act2 = action.reshape(SB, 128)
    out_dom, nh_right = _tc_call(holding, act2, dom)
    nh_left = _sc_call(action, holding, dom)
    next_holding = jnp.concatenate([nh_left, nh_right.reshape(-1)])
    return next_holding, out_dom.reshape(-1)


# speedup vs baseline: 1.0761x; 1.0550x over previous
import functools

import jax
import jax.numpy as jnp
from jax import lax
from jax.experimental import pallas as pl
from jax.experimental.pallas import tpu as pltpu
from jax.experimental.pallas import tpu_sc as plsc

C = 4096
NC, NS, L = 2, 16, 16
SB = 32
W = 128
NSCB = 16
NG = W // L
R = 128
RSPLIT = 2
SPAN = C // RSPLIT
HALF = NSCB * W


def _sc_body(act_hbm, hold_hbm, dom_hbm, outhold_hbm,
             act_v, hold_v, nh_v, part_v, dbuf, shared,
             sem_in0, sem_in1):
    core = lax.axis_index("c")
    sub = lax.axis_index("s")
    sem_in = (sem_in0, sem_in1)
    band = core * (NSCB // NC) + sub // RSPLIT
    rh = sub % RSPLIT
    n0 = band * W
    m_base = rh * SPAN

    pltpu.sync_copy(act_hbm.at[pl.ds(n0, W)], act_v)
    pltpu.sync_copy(hold_hbm.at[:], hold_v.at[pl.ds(0, C)])

    a_g = [act_v[pl.ds(g * L, L)] for g in range(NG)]

    zero = jnp.zeros((L,), jnp.float32)
    carry = tuple(zero for _ in range(3 * NG))

    def in_copy(j, p):
        return pltpu.async_copy(
            dom_hbm.at[pl.ds(m_base + j * R, R), band], dbuf.at[p], sem_in[p])

    in_copy(0, 0)
    in_copy(1, 1)

    RU = 8
    NCH = R // RU
    NBH = SPAN // R

    def pair_body(i, carry):
        for p in (0, 1):
            j = 2 * i + p
            db = dbuf.at[p]
            pltpu.make_async_copy(
                dom_hbm.at[pl.ds(0, R), band], db, sem_in[p]).wait()
            m0 = m_base + j * R

            def chunk_body(cc, t, db=db, m0=m0):
                mb = cc * RU
                hvec = hold_v[pl.ds(m0 + mb, L)]
                t = list(t)
                for k in range(RU):
                    hv = jnp.full((L,), hvec[k], jnp.float32)
                    for g in range(NG):
                        d = db[mb + k, pl.ds(g * L, L)]
                        pr = d * hv
                        t0, t1, t2 = t[3 * g], t[3 * g + 1], t[3 * g + 2]
                        n0v = jnp.maximum(t0, pr)
                        r1 = jnp.minimum(t0, pr)
                        n1v = jnp.maximum(t1, r1)
                        r2 = jnp.minimum(t1, r1)
                        n2v = jnp.maximum(t2, r2)
                        t[3 * g], t[3 * g + 1], t[3 * g + 2] = n0v, n1v, n2v
                return tuple(t)

            carry = lax.fori_loop(0, NCH, chunk_body, tuple(carry))

            @pl.when(j + 2 < NBH)
            def _():
                in_copy(j + 2, p)

        return carry

    carry = lax.fori_loop(0, NBH // 2, pair_body, carry)

    for r in range(3):
        for g in range(NG):
            part_v[r, pl.ds(g * L, L)] = carry[3 * g + r]
    pltpu.sync_copy(part_v, shared.at[sub])
    plsc.subcore_barrier()

    @pl.when(rh == 0)
    def _():
        t = list(carry)
        for q in range(1, RSPLIT):
            pltpu.sync_copy(shared.at[sub + q], part_v)
            for g in range(NG):
                t0, t1, t2 = t[3 * g], t[3 * g + 1], t[3 * g + 2]
                for r in range(3):
                    pr = part_v[r, pl.ds(g * L, L)]
                    t0n = jnp.maximum(t0, pr)
                    r1 = jnp.minimum(t0, pr)
                    t1n = jnp.maximum(t1, r1)
                    r2 = jnp.minimum(t1, r1)
                    t2n = jnp.maximum(t2, r2)
                    t0, t1, t2 = t0n, t1n, t2n
                t[3 * g], t[3 * g + 1], t[3 * g + 2] = t0, t1, t2
        for g in range(NG):
            v0 = t[3 * g] * a_g[g]
            v1 = t[3 * g + 1] * a_g[g]
            v2 = t[3 * g + 2] * a_g[g]
            nh_v[pl.ds(g * L, L)] = (
                1.0 - (1.0 - v0) * (1.0 - v1) * (1.0 - v2))
        pltpu.sync_copy(nh_v, outhold_hbm.at[pl.ds(n0, W)])


_sc_call = functools.partial(
    pl.kernel,
    out_type=jax.ShapeDtypeStruct((HALF,), jnp.float32),
    mesh=plsc.VectorSubcoreMesh(
        core_axis_name="c", subcore_axis_name="s", num_cores=NC,
        num_subcores=NS),
    scratch_types=[
        pltpu.VMEM((W,), jnp.float32),
        pltpu.VMEM((C + L,), jnp.float32),
        pltpu.VMEM((W,), jnp.float32),
        pltpu.VMEM((3, W), jnp.float32),
        pltpu.VMEM((2, R, W), jnp.float32),
        pltpu.VMEM_SHARED((NS, 3, W), jnp.float32),
        pltpu.SemaphoreType.DMA,
        pltpu.SemaphoreType.DMA,
    ],
)(_sc_body)


TBM = 256
HS = SB - NSCB


def _tc_body(hold_smem, act_ref, dom_ref, out_ref, nh_ref, t_ref):
    gi = pl.program_id(0)
    i0 = gi * TBM
    A = 1.0 - act_ref[...]

    @pl.when(gi == 0)
    def _():
        t_ref[...] = jnp.zeros((3, HS, 128), jnp.float32)

    def slab(i, t):
        t0, t1, t2 = t
        h = hold_smem[i0 + i]
        d = dom_ref[i]
        p1 = d * A
        p2 = d * (1.0 - h)
        out_ref[i] = p1 + p2 - p1 * p2
        pr = d[NSCB:, :] * h
        n0v = jnp.maximum(t0, pr)
        r1 = jnp.minimum(t0, pr)
        n1v = jnp.maximum(t1, r1)
        r2 = jnp.minimum(t1, r1)
        n2v = jnp.maximum(t2, r2)
        return (n0v, n1v, n2v)

    t = lax.fori_loop(0, TBM, slab, (t_ref[0], t_ref[1], t_ref[2]))
    t_ref[0], t_ref[1], t_ref[2] = t

    @pl.when(gi == C // TBM - 1)
    def _():
        a = act_ref[NSCB:, :]
        v0 = t_ref[0] * a
        v1 = t_ref[1] * a
        v2 = t_ref[2] * a
        nh_ref[...] = 1.0 - (1.0 - v0) * (1.0 - v1) * (1.0 - v2)


_tc_call = pl.pallas_call(
    _tc_body,
    grid=(C // TBM,),
    in_specs=[
        pl.BlockSpec(memory_space=pltpu.SMEM),
        pl.BlockSpec((SB, 128), lambda i: (0, 0)),
        pl.BlockSpec((TBM, SB, 128), lambda i: (i, 0, 0)),
    ],
    out_specs=[
        pl.BlockSpec((TBM, SB, 128), lambda i: (i, 0, 0)),
        pl.BlockSpec((HS, 128), lambda i: (0, 0)),
    ],
    out_shape=[
        jax.ShapeDtypeStruct((C, SB, 128), jnp.float32),
        jax.ShapeDtypeStruct((HS, 128), jnp.float32),
    ],
    scratch_shapes=[pltpu.VMEM((3, HS, 128), jnp.float32)],
)


def kernel(action, holding, dominos):
    dom = dominos.reshape(C, SB, 128)
    act2 = action.reshape(SB, 128)
    out_dom, nh_right = _tc_call(holding, act2, dom)
    nh_left = _sc_call(action, holding, dom)
    next_holding = jnp.concatenate([nh_left, nh_right.reshape(-1)])
    return next_holding, out_dom.reshape(-1)


# speedup vs baseline: 1.1239x; 1.0445x over previous
import functools

import jax
import jax.numpy as jnp
from jax import lax
from jax.experimental import pallas as pl
from jax.experimental.pallas import tpu as pltpu
from jax.experimental.pallas import tpu_sc as plsc

C = 4096
NC, NS, L = 2, 16, 16
SB = 32
W = 128
NSCB = 16
NG = W // L
R = 128
RSPLIT = 2
SPAN = C // RSPLIT
HALF = NSCB * W


def _sc_body(act_hbm, hold_hbm, dom_hbm, outhold_hbm,
             act_v, hold_v, nh_v, part_v, dbuf, shared,
             sem_in0, sem_in1):
    core = lax.axis_index("c")
    sub = lax.axis_index("s")
    sem_in = (sem_in0, sem_in1)
    band = core * (NSCB // NC) + sub // RSPLIT
    rh = sub % RSPLIT
    n0 = band * W
    m_base = rh * SPAN

    pltpu.sync_copy(act_hbm.at[pl.ds(n0, W)], act_v)
    pltpu.sync_copy(hold_hbm.at[:], hold_v.at[pl.ds(0, C)])

    a_g = [act_v[pl.ds(g * L, L)] for g in range(NG)]

    zero = jnp.zeros((L,), jnp.float32)
    carry = tuple(zero for _ in range(3 * NG))

    def in_copy(j, p):
        return pltpu.async_copy(
            dom_hbm.at[pl.ds(m_base + j * R, R), band], dbuf.at[p], sem_in[p])

    in_copy(0, 0)
    in_copy(1, 1)

    RU = 8
    NCH = R // RU
    NBH = SPAN // R

    def pair_body(i, carry):
        for p in (0, 1):
            j = 2 * i + p
            db = dbuf.at[p]
            pltpu.make_async_copy(
                dom_hbm.at[pl.ds(0, R), band], db, sem_in[p]).wait()
            m0 = m_base + j * R

            def chunk_body(cc, t, db=db, m0=m0):
                mb = cc * RU
                hvec = hold_v[pl.ds(m0 + mb, L)]
                t = list(t)
                for k in range(RU):
                    hv = jnp.full((L,), hvec[k], jnp.float32)
                    for g in range(NG):
                        d = db[mb + k, pl.ds(g * L, L)]
                        pr = d * hv
                        t0, t1, t2 = t[3 * g], t[3 * g + 1], t[3 * g + 2]
                        n0v = jnp.maximum(t0, pr)
                        r1 = jnp.minimum(t0, pr)
                        n1v = jnp.maximum(t1, r1)
                        r2 = jnp.minimum(t1, r1)
                        n2v = jnp.maximum(t2, r2)
                        t[3 * g], t[3 * g + 1], t[3 * g + 2] = n0v, n1v, n2v
                return tuple(t)

            carry = lax.fori_loop(0, NCH, chunk_body, tuple(carry))

            @pl.when(j + 2 < NBH)
            def _():
                in_copy(j + 2, p)

        return carry

    carry = lax.fori_loop(0, NBH // 2, pair_body, carry)

    for r in range(3):
        for g in range(NG):
            part_v[r, pl.ds(g * L, L)] = carry[3 * g + r]
    pltpu.sync_copy(part_v, shared.at[sub])
    plsc.subcore_barrier()

    @pl.when(rh == 0)
    def _():
        t = list(carry)
        for q in range(1, RSPLIT):
            pltpu.sync_copy(shared.at[sub + q], part_v)
            for g in range(NG):
                t0, t1, t2 = t[3 * g], t[3 * g + 1], t[3 * g + 2]
                for r in range(3):
                    pr = part_v[r, pl.ds(g * L, L)]
                    t0n = jnp.maximum(t0, pr)
                    r1 = jnp.minimum(t0, pr)
                    t1n = jnp.maximum(t1, r1)
                    r2 = jnp.minimum(t1, r1)
                    t2n = jnp.maximum(t2, r2)
                    t0, t1, t2 = t0n, t1n, t2n
                t[3 * g], t[3 * g + 1], t[3 * g + 2] = t0, t1, t2
        for g in range(NG):
            v0 = t[3 * g] * a_g[g]
            v1 = t[3 * g + 1] * a_g[g]
            v2 = t[3 * g + 2] * a_g[g]
            nh_v[pl.ds(g * L, L)] = (
                1.0 - (1.0 - v0) * (1.0 - v1) * (1.0 - v2))
        pltpu.sync_copy(nh_v, outhold_hbm.at[pl.ds(n0, W)])


_sc_call = functools.partial(
    pl.kernel,
    out_type=jax.ShapeDtypeStruct((HALF,), jnp.float32),
    mesh=plsc.VectorSubcoreMesh(
        core_axis_name="c", subcore_axis_name="s", num_cores=NC,
        num_subcores=NS),
    scratch_types=[
        pltpu.VMEM((W,), jnp.float32),
        pltpu.VMEM((C + L,), jnp.float32),
        pltpu.VMEM((W,), jnp.float32),
        pltpu.VMEM((3, W), jnp.float32),
        pltpu.VMEM((2, R, W), jnp.float32),
        pltpu.VMEM_SHARED((NS, 3, W), jnp.float32),
        pltpu.SemaphoreType.DMA,
        pltpu.SemaphoreType.DMA,
    ],
)(_sc_body)


TBM = 512
HS = SB - NSCB


def _tc_body(hold_smem, act_ref, dom_ref, out_ref, nh_ref, t_ref):
    gi = pl.program_id(0)
    i0 = gi * TBM
    A = 1.0 - act_ref[...]

    @pl.when(gi == 0)
    def _():
        t_ref[...] = jnp.zeros((3, HS, 128), jnp.float32)

    def slab(i, t):
        t0, t1, t2 = t
        h = hold_smem[i0 + i]
        d = dom_ref[i]
        p1 = d * A
        p2 = d * (1.0 - h)
        out_ref[i] = p1 + p2 - p1 * p2
        pr = d[NSCB:, :] * h
        n0v = jnp.maximum(t0, pr)
        r1 = jnp.minimum(t0, pr)
        n1v = jnp.maximum(t1, r1)
        r2 = jnp.minimum(t1, r1)
        n2v = jnp.maximum(t2, r2)
        return (n0v, n1v, n2v)

    t = lax.fori_loop(0, TBM, slab, (t_ref[0], t_ref[1], t_ref[2]))
    t_ref[0], t_ref[1], t_ref[2] = t

    @pl.when(gi == C // TBM - 1)
    def _():
        a = act_ref[NSCB:, :]
        v0 = t_ref[0] * a
        v1 = t_ref[1] * a
        v2 = t_ref[2] * a
        nh_ref[...] = 1.0 - (1.0 - v0) * (1.0 - v1) * (1.0 - v2)


_tc_call = pl.pallas_call(
    _tc_body,
    grid=(C // TBM,),
    in_specs=[
        pl.BlockSpec(memory_space=pltpu.SMEM),
        pl.BlockSpec((SB, 128), lambda i: (0, 0)),
        pl.BlockSpec((TBM, SB, 128), lambda i: (i, 0, 0)),
    ],
    out_specs=[
        pl.BlockSpec((TBM, SB, 128), lambda i: (i, 0, 0)),
        pl.BlockSpec((HS, 128), lambda i: (0, 0)),
    ],
    out_shape=[
        jax.ShapeDtypeStruct((C, SB, 128), jnp.float32),
        jax.ShapeDtypeStruct((HS, 128), jnp.float32),
    ],
    scratch_shapes=[pltpu.VMEM((3, HS, 128), jnp.float32)],
)


def kernel(action, holding, dominos):
    dom = dominos.reshape(C, SB, 128)
    act2 = action.reshape(SB, 128)
    out_dom, nh_right = _tc_call(holding, act2, dom)
    nh_left = _sc_call(action, holding, dom)
    next_holding = jnp.concatenate([nh_left, nh_right.reshape(-1)])
    return next_holding, out_dom.reshape(-1)
